# baseline (speedup 1.0000x reference)
import functools

import jax
import jax.numpy as jnp
from jax.experimental import pallas as pl
from jax.experimental.pallas import tpu as pltpu

_B, _F, _N, _E = 8, 1024, 2048, 128
_CN, _CF = 512, 256
_NEG = -1e30


def _row_dot(a, b):
    return jax.lax.dot_general(a, b, (((1,), (1,)), ((), ())),
                               preferred_element_type=jnp.float32)


def _body(rel_ref, arg1_ref, fr_ref, fa1_ref, fa2_ref, nbf_ref, ents_ref,
          nbe_ref, w0_ref, wp0_ref, w1_ref, wp1_ref, out_ref):
    rel = rel_ref[0]
    src = arg1_ref[0]
    nbf = nbf_ref[0, 0, 0]
    nbe = nbe_ref[0, 0, 0]
    n_tiles = (nbe + _CN - 1) // _CN
    f_tiles = (nbf + _CF - 1) // _CF

    inv_e = 1.0 / _E
    half = 0.5 * inv_e
    ones = jnp.ones((1, _E), jnp.float32)
    cf_iota = jax.lax.broadcasted_iota(jnp.int32, (1, _CF), 1)
    cn_iota = jax.lax.broadcasted_iota(jnp.int32, (_CN, 1), 0)

    src2 = jnp.sum(src * src, axis=(0, 1), keepdims=True)

    def run_rule(w_ref, wp_ref, fa_src_ref, fa_ent_ref):
        hop = jnp.dot(rel, w_ref[...], preferred_element_type=jnp.float32)
        hop2 = jnp.sum(hop * hop, axis=(0, 1), keepdims=True)

        def n_loop(ni, m):
            n0 = ni * _CN
            ec = ents_ref[0, pl.ds(n0, _CN), :]
            ecb = ec.astype(jnp.bfloat16)
            xnh = jnp.sum(ec * ec, axis=1, keepdims=True) * half
            nmask = (cn_iota + n0) < nbe

            def f_loop(fi, m2):
                f0 = fi * _CF
                frc = fr_ref[0, pl.ds(f0, _CF), :]
                fsc = fa_src_ref[0, pl.ds(f0, _CF), :]
                fec = fa_ent_ref[0, pl.ds(f0, _CF), :]
                frn = _row_dot(ones, frc * frc)
                fsn = _row_dot(ones, fsc * fsc)
                yn = _row_dot(ones, fec * fec)
                d2r = hop2 + frn - 2.0 * _row_dot(hop, frc)
                d2s = src2 + fsn - 2.0 * _row_dot(src, fsc)
                c = -(d2r + d2s + yn) * half
                c = jnp.where((cf_iota + f0) < nbf, c, _NEG)
                xy = jax.lax.dot_general(
                    ecb, (fec * inv_e).astype(jnp.bfloat16),
                    (((1,), (1,)), ((), ())),
                    preferred_element_type=jnp.float32)
                val = jnp.max(xy + c, axis=1, keepdims=True)
                s = jnp.where(nmask, val - xnh, _NEG)
                return jnp.maximum(m2, jnp.max(s, axis=(0, 1), keepdims=True))

            return jax.lax.fori_loop(0, f_tiles, f_loop, m)

        m = jax.lax.fori_loop(0, n_tiles, n_loop,
                              jnp.full((1, 1), _NEG, jnp.float32))
        logit = jnp.sum(rel * wp_ref[...], axis=(0, 1), keepdims=True)
        prior = jax.nn.sigmoid(logit)
        return jnp.minimum(jnp.exp(m), prior)

    r0 = run_rule(w0_ref, wp0_ref, fa1_ref, fa2_ref)
    r1 = run_rule(w1_ref, wp1_ref, fa2_ref, fa1_ref)
    out_ref[0] = jnp.broadcast_to(jnp.maximum(r0, r1), (1, _E))


@jax.jit
def kernel(rel, arg1, arg2, fact_rel, fact_arg1, fact_arg2, nb_facts,
           entity_embeddings, nb_entities, W_hop_0, w_prior_0, W_hop_1,
           w_prior_1):
    del arg2
    nbf = nb_facts.reshape(_B, 1, 1)
    nbe = nb_entities.reshape(_B, 1, 1)
    wp0 = w_prior_0.reshape(1, _E)
    wp1 = w_prior_1.reshape(1, _E)
    rel3 = rel.reshape(_B, 1, _E)
    arg13 = arg1.reshape(_B, 1, _E)

    vec = pl.BlockSpec((1, 1, _E), lambda b: (b, 0, 0))
    facts = pl.BlockSpec((1, _F, _E), lambda b: (b, 0, 0))
    smem = pl.BlockSpec((1, 1, 1), lambda b: (b, 0, 0),
                        memory_space=pltpu.SMEM)
    const2 = pl.BlockSpec((_E, _E), lambda b: (0, 0))
    const_row = pl.BlockSpec((1, _E), lambda b: (0, 0))

    out = pl.pallas_call(
        _body,
        grid=(_B,),
        in_specs=[vec, vec, facts, facts, facts, smem,
                  pl.BlockSpec((1, _N, _E), lambda b: (b, 0, 0)), smem,
                  const2, const_row, const2, const_row],
        out_specs=pl.BlockSpec((1, 1, _E), lambda b: (b, 0, 0)),
        out_shape=jax.ShapeDtypeStruct((_B, 1, _E), jnp.float32),
        compiler_params=pltpu.CompilerParams(
            dimension_semantics=("arbitrary",)),
    )(rel3, arg13, fact_rel, fact_arg1, fact_arg2, nbf, entity_embeddings,
      nbe, W_hop_0, wp0, W_hop_1, wp1)
    return out[:, 0, 0]


# speedup vs baseline: 1.3492x; 1.3492x over previous
import functools

import jax
import jax.numpy as jnp
from jax.experimental import pallas as pl
from jax.experimental.pallas import tpu as pltpu

_B, _F, _N, _E = 8, 1024, 2048, 128
_CF = 256
_FC = _F // _CF
_NEG = -1e30


def _row_dot(a, b):
    return jax.lax.dot_general(a, b, (((1,), (1,)), ((), ())),
                               preferred_element_type=jnp.float32)


def _body(rel_ref, arg1_ref, fr_ref, fa1_ref, fa2_ref, nbf_ref, ents_ref,
          nbe_ref, w0_ref, wp0_ref, w1_ref, wp1_ref, out_ref,
          ents_scr, sbias_scr, m0_scr, m1_scr):
    fc = pl.program_id(1)
    f0 = fc * _CF
    nbf = nbf_ref[0, 0, 0]
    nbe = nbe_ref[0, 0, 0]
    rel = rel_ref[0]
    src = arg1_ref[0]

    inv_e = 1.0 / _E
    half = 0.5 * inv_e

    @pl.when(fc == 0)
    def _init():
        ents = ents_ref[0]
        ents_scr[...] = ents.astype(jnp.bfloat16)
        xnh = jnp.sum(ents * ents, axis=1, keepdims=True) * half
        n_iota = jax.lax.broadcasted_iota(jnp.int32, (_N, 1), 0)
        sbias_scr[...] = jnp.where(n_iota < nbe, -xnh, _NEG)
        m0_scr[...] = jnp.full((1, _E), _NEG, jnp.float32)
        m1_scr[...] = jnp.full((1, _E), _NEG, jnp.float32)

    @pl.when(f0 < nbf)
    def _compute():
        lhsb = ents_scr[...]
        sbias = sbias_scr[...]
        frc = fr_ref[0]
        fa1c = fa1_ref[0]
        fa2c = fa2_ref[0]
        ones = jnp.ones((1, _E), jnp.float32)
        cf_iota = jax.lax.broadcasted_iota(jnp.int32, (1, _CF), 1)
        fmask = (cf_iota + f0) < nbf
        src2 = jnp.sum(src * src, axis=(0, 1), keepdims=True)
        frn = _row_dot(ones, frc * frc)

        def one(w_ref, fa_src, fa_ent, m_scr):
            hop = jnp.dot(rel, w_ref[...],
                          preferred_element_type=jnp.float32)
            hop2 = jnp.sum(hop * hop, axis=(0, 1), keepdims=True)
            fsn = _row_dot(ones, fa_src * fa_src)
            yn = _row_dot(ones, fa_ent * fa_ent)
            d2r = hop2 + frn - 2.0 * _row_dot(hop, frc)
            d2s = src2 + fsn - 2.0 * _row_dot(src, fa_src)
            c = -(d2r + d2s + yn) * half
            c = jnp.where(fmask, c, _NEG)
            xy = jax.lax.dot_general(
                lhsb, (fa_ent * inv_e).astype(jnp.bfloat16),
                (((1,), (1,)), ((), ())),
                preferred_element_type=jnp.float32)
            z = jnp.max(xy + sbias, axis=0, keepdims=True)
            mc = jnp.max(z + c, axis=(0, 1), keepdims=True)
            m_scr[...] = jnp.maximum(m_scr[...], mc)

        one(w0_ref, fa1c, fa2c, m0_scr)
        one(w1_ref, fa2c, fa1c, m1_scr)

    @pl.when(fc == _FC - 1)
    def _finalize():
        def prior(wp_ref):
            logit = jnp.sum(rel * wp_ref[...], axis=(0, 1), keepdims=True)
            return jax.nn.sigmoid(logit)

        r0 = jnp.minimum(jnp.exp(m0_scr[...]), prior(wp0_ref))
        r1 = jnp.minimum(jnp.exp(m1_scr[...]), prior(wp1_ref))
        out_ref[0] = jnp.maximum(r0, r1)


@jax.jit
def kernel(rel, arg1, arg2, fact_rel, fact_arg1, fact_arg2, nb_facts,
           entity_embeddings, nb_entities, W_hop_0, w_prior_0, W_hop_1,
           w_prior_1):
    del arg2
    nbf = nb_facts.reshape(_B, 1, 1)
    nbe = nb_entities.reshape(_B, 1, 1)
    wp0 = w_prior_0.reshape(1, _E)
    wp1 = w_prior_1.reshape(1, _E)
    rel3 = rel.reshape(_B, 1, _E)
    arg13 = arg1.reshape(_B, 1, _E)

    vec = pl.BlockSpec((1, 1, _E), lambda b, fc: (b, 0, 0))
    facts = pl.BlockSpec((1, _CF, _E), lambda b, fc: (b, fc, 0))
    smem = pl.BlockSpec((1, 1, 1), lambda b, fc: (b, 0, 0),
                        memory_space=pltpu.SMEM)
    const2 = pl.BlockSpec((_E, _E), lambda b, fc: (0, 0))
    const_row = pl.BlockSpec((1, _E), lambda b, fc: (0, 0))

    out = pl.pallas_call(
        _body,
        grid=(_B, _FC),
        in_specs=[vec, vec, facts, facts, facts, smem,
                  pl.BlockSpec((1, _N, _E), lambda b, fc: (b, 0, 0)), smem,
                  const2, const_row, const2, const_row],
        out_specs=pl.BlockSpec((1, 1, _E), lambda b, fc: (b, 0, 0)),
        out_shape=jax.ShapeDtypeStruct((_B, 1, _E), jnp.float32),
        scratch_shapes=[
            pltpu.VMEM((_N, _E), jnp.bfloat16),
            pltpu.VMEM((_N, 1), jnp.float32),
            pltpu.VMEM((1, _E), jnp.float32),
            pltpu.VMEM((1, _E), jnp.float32),
        ],
        compiler_params=pltpu.CompilerParams(
            dimension_semantics=("arbitrary", "arbitrary")),
    )(rel3, arg13, fact_rel, fact_arg1, fact_arg2, nbf, entity_embeddings,
      nbe, W_hop_0, wp0, W_hop_1, wp1)
    return out[:, 0, 0]


# speedup vs baseline: 1.6932x; 1.2550x over previous
import functools

import jax
import jax.numpy as jnp
from jax.experimental import pallas as pl
from jax.experimental.pallas import tpu as pltpu

_B, _F, _N, _E = 8, 1024, 2048, 128
_NEG = -1e30


def _row_dot(a, b):
    return jax.lax.dot_general(a, b, (((1,), (1,)), ((), ())),
                               preferred_element_type=jnp.float32)


def _rule(hop, src, fr, fa_src, fa_ent, ents, xn_half, nbf, nbe, f_iota, n_iota):
    inv_e = 1.0 / _E
    half = 0.5 * inv_e
    ones = jnp.ones((1, _E), jnp.float32)

    frn = _row_dot(ones, fr * fr)
    fsn = _row_dot(ones, fa_src * fa_src)
    yn = _row_dot(ones, fa_ent * fa_ent)
    hop2 = jnp.sum(hop * hop, axis=(0, 1), keepdims=True)
    src2 = jnp.sum(src * src, axis=(0, 1), keepdims=True)
    d2r = hop2 + frn - 2.0 * _row_dot(hop, fr)
    d2s = src2 + fsn - 2.0 * _row_dot(src, fa_src)
    c = -(d2r + d2s + yn) * half
    c = jnp.where(f_iota < nbf, c, _NEG)

    xy = jax.lax.dot_general(ents.astype(jnp.bfloat16),
                             (fa_ent * inv_e).astype(jnp.bfloat16),
                             (((1,), (1,)), ((), ())),
                             preferred_element_type=jnp.float32)
    val = jnp.max(xy + c, axis=1, keepdims=True)
    s = val - xn_half
    s = jnp.where(n_iota < nbe, s, _NEG)
    return jnp.max(s, axis=(0, 1), keepdims=True)


def _body(rel_ref, arg1_ref, fr_ref, fa1_ref, fa2_ref, nbf_ref, ents_ref,
          nbe_ref, w0_ref, wp0_ref, w1_ref, wp1_ref, out_ref):
    rel = rel_ref[0]
    src = arg1_ref[0]
    fr = fr_ref[0]
    fa1 = fa1_ref[0]
    fa2 = fa2_ref[0]
    ents = ents_ref[0]
    nbf = nbf_ref[0, 0, 0]
    nbe = nbe_ref[0, 0, 0]

    f_iota = jax.lax.broadcasted_iota(jnp.int32, (1, _F), 1)
    n_iota = jax.lax.broadcasted_iota(jnp.int32, (_N, 1), 0)
    xn_half = jnp.sum(ents * ents, axis=1, keepdims=True) * (0.5 / _E)

    def one(w_ref, wp_ref, fa_src, fa_ent):
        hop = jnp.dot(rel, w_ref[...], preferred_element_type=jnp.float32)
        m = _rule(hop, src, fr, fa_src, fa_ent, ents, xn_half, nbf, nbe,
                  f_iota, n_iota)
        logit = jnp.sum(rel * wp_ref[...], axis=(0, 1), keepdims=True)
        prior = jax.nn.sigmoid(logit)
        return jnp.minimum(jnp.exp(m), prior)

    r0 = one(w0_ref, wp0_ref, fa1, fa2)
    r1 = one(w1_ref, wp1_ref, fa2, fa1)
    out_ref[0] = jnp.broadcast_to(jnp.maximum(r0, r1), (1, _E))


@jax.jit
def kernel(rel, arg1, arg2, fact_rel, fact_arg1, fact_arg2, nb_facts,
           entity_embeddings, nb_entities, W_hop_0, w_prior_0, W_hop_1,
           w_prior_1):
    del arg2
    nbf = nb_facts.reshape(_B, 1, 1)
    nbe = nb_entities.reshape(_B, 1, 1)
    wp0 = w_prior_0.reshape(1, _E)
    wp1 = w_prior_1.reshape(1, _E)
    rel3 = rel.reshape(_B, 1, _E)
    arg13 = arg1.reshape(_B, 1, _E)

    vec = pl.BlockSpec((1, 1, _E), lambda b: (b, 0, 0))
    facts = pl.BlockSpec((1, _F, _E), lambda b: (b, 0, 0))
    smem = pl.BlockSpec((1, 1, 1), lambda b: (b, 0, 0),
                        memory_space=pltpu.SMEM)
    const2 = pl.BlockSpec((_E, _E), lambda b: (0, 0))
    const_row = pl.BlockSpec((1, _E), lambda b: (0, 0))

    out = pl.pallas_call(
        _body,
        grid=(_B,),
        in_specs=[vec, vec, facts, facts, facts, smem,
                  pl.BlockSpec((1, _N, _E), lambda b: (b, 0, 0)), smem,
                  const2, const_row, const2, const_row],
        out_specs=pl.BlockSpec((1, 1, _E), lambda b: (b, 0, 0)),
        out_shape=jax.ShapeDtypeStruct((_B, 1, _E), jnp.float32),
        compiler_params=pltpu.CompilerParams(
            dimension_semantics=("parallel",),
            vmem_limit_bytes=100 * 1024 * 1024),
    )(rel3, arg13, fact_rel, fact_arg1, fact_arg2, nbf, entity_embeddings,
      nbe, W_hop_0, wp0, W_hop_1, wp1)
    return out[:, 0, 0]
